# Initial kernel scaffold; baseline (speedup 1.0000x reference)
import jax
import jax.numpy as jnp
from jax.experimental import pallas as pl


def kernel(notes, raters, note_emb, rater_emb, note_bias, rater_bias, rater_rep, global_bias):
    raise NotImplementedError("write your pallas kernel here")


# speedup vs baseline: 1.0827x; 1.0827x over previous
import functools

import jax
import jax.numpy as jnp
from jax import lax
from jax.experimental import pallas as pl
from jax.experimental.pallas import tpu as pltpu
from jax.experimental.pallas import tpu_sc as plsc

NUM_ROWS = 1000000
BATCH = 16384

NC = 2
NS = 16
NW = NC * NS
BPW = BATCH // NW
CHUNK = 128
NCH = BPW // CHUNK
LANES = 16


def _mf_kernel(notes_hbm, raters_hbm, note_emb, rater_emb, note_bias,
               rater_bias, rater_rep, gb_hbm, out_hbm,
               idx_n, idx_r, ne_v, re_v, nb_v, rb_v, rr_v, out_v, gb_v, sem):
  wid = lax.axis_index("s") * NC + lax.axis_index("c")
  base = wid * BPW

  for j in range(NCH):
    pltpu.sync_copy(notes_hbm.at[pl.ds(base + j * CHUNK, CHUNK)], idx_n.at[j])
    pltpu.sync_copy(raters_hbm.at[pl.ds(base + j * CHUNK, CHUNK)], idx_r.at[j])
  pltpu.sync_copy(gb_hbm, gb_v)

  copies = []
  for j in range(NCH):
    copies.append(pltpu.async_copy(note_emb.at[idx_n.at[j]], ne_v.at[j], sem))
    copies.append(pltpu.async_copy(rater_emb.at[idx_r.at[j]], re_v.at[j], sem))
    copies.append(pltpu.async_copy(note_bias.at[idx_n.at[j]], nb_v.at[j], sem))
    copies.append(pltpu.async_copy(rater_bias.at[idx_r.at[j]], rb_v.at[j], sem))
    copies.append(pltpu.async_copy(rater_rep.at[idx_r.at[j]], rr_v.at[j], sem))
  for c in copies:
    c.wait()

  gb = gb_v[...]
  for j in range(NCH):
    for k in range(CHUNK // LANES):
      sl = pl.ds(k * LANES, LANES)
      p = (ne_v[j, sl] * re_v[j, sl]
           + nb_v[j, sl] * rr_v[j, sl]
           + rb_v[j, sl] + gb)
      out_v[j, sl] = 1.0 / (1.0 + jnp.exp(-p))

  for j in range(NCH):
    pltpu.sync_copy(out_v.at[j], out_hbm.at[pl.ds(base + j * CHUNK, CHUNK)])


@jax.jit
def _run(notes, raters, note_emb, rater_emb, note_bias, rater_bias,
         rater_rep, gb16):
  mesh = plsc.VectorSubcoreMesh(core_axis_name="c", subcore_axis_name="s")
  f32 = jnp.float32
  scratch = [
      pltpu.VMEM((NCH, CHUNK), jnp.int32),
      pltpu.VMEM((NCH, CHUNK), jnp.int32),
      pltpu.VMEM((NCH, CHUNK), f32),
      pltpu.VMEM((NCH, CHUNK), f32),
      pltpu.VMEM((NCH, CHUNK), f32),
      pltpu.VMEM((NCH, CHUNK), f32),
      pltpu.VMEM((NCH, CHUNK), f32),
      pltpu.VMEM((NCH, CHUNK), f32),
      pltpu.VMEM((LANES,), f32),
      pltpu.SemaphoreType.DMA,
  ]
  run = pl.kernel(
      _mf_kernel,
      out_type=jax.ShapeDtypeStruct((BATCH,), f32),
      mesh=mesh,
      scratch_types=scratch,
  )
  return run(notes, raters, note_emb, rater_emb, note_bias, rater_bias,
             rater_rep, gb16)


def kernel(notes, raters, note_emb, rater_emb, note_bias, rater_bias,
           rater_rep, global_bias):
  gb16 = jnp.broadcast_to(jnp.reshape(global_bias, (1,)), (LANES,))
  out = _run(
      notes.astype(jnp.int32), raters.astype(jnp.int32),
      note_emb.reshape(NUM_ROWS), rater_emb.reshape(NUM_ROWS),
      note_bias.reshape(NUM_ROWS), rater_bias.reshape(NUM_ROWS),
      rater_rep.reshape(NUM_ROWS), gb16)
  return out.reshape(BATCH, 1)
